# baseline (speedup 1.0000x reference)
import math

import jax
import jax.numpy as jnp
from jax.experimental import pallas as pl
from jax.experimental.pallas import tpu as pltpu

_GOLDEN_CENTER = 1 / math.e
_GOLDEN_LOWER = 0.5 - math.log(4 / 3)
_GOLDEN_UPPER = 0.5


def _moe_body(temp_ref, x_ref, wr_ref, wg_ref, wu_ref, wd_ref, out_ref,
              xb_s, hb_s, w_s):
    k = pl.program_id(0)
    E = wr_ref.shape[0]
    FS = wg_ref.shape[1]
    DS = wd_ref.shape[1]
    dn = (((1,), (1,)), ((), ()))

    @pl.when(k == 0)
    def _router_and_xcast():
        xc = x_ref[...]
        xb_s[...] = xc.astype(jnp.bfloat16)
        logits = jax.lax.dot_general(
            xc, wr_ref[...], dn,
            preferred_element_type=jnp.float32)
        inhib = jax.nn.sigmoid(logits / temp_ref[0])
        in_zone = jnp.logical_and(inhib >= _GOLDEN_LOWER, inhib <= _GOLDEN_UPPER)
        dist = jnp.abs(inhib - _GOLDEN_CENTER)
        w = jnp.exp(-dist / 0.1) * in_zone.astype(jnp.float32)
        wsum = jnp.sum(w, axis=1, keepdims=True)
        no_expert = wsum < 1e-8
        fb = jnp.exp(-dist / 0.3)
        idx = jax.lax.broadcasted_iota(jnp.int32, fb.shape, 1)
        m1 = jnp.max(fb, axis=1, keepdims=True)
        i1 = jnp.min(jnp.where(fb == m1, idx, E), axis=1, keepdims=True)
        mask1 = idx == i1
        fb2 = jnp.where(mask1, -jnp.inf, fb)
        m2 = jnp.max(fb2, axis=1, keepdims=True)
        i2 = jnp.min(jnp.where(fb2 == m2, idx, E), axis=1, keepdims=True)
        fbm = jnp.logical_or(mask1, idx == i2).astype(jnp.float32)
        fb_w = fb * fbm
        fb_w = fb_w / jnp.maximum(jnp.sum(fb_w, axis=1, keepdims=True), 1e-8)
        w = jnp.where(no_expert, fb_w, w)
        w = w / jnp.maximum(jnp.sum(w, axis=1, keepdims=True), 1e-8)
        w_s[...] = w

    par = (k // 2) % 2
    half = k % 2

    xb = xb_s[...]
    gate = jax.lax.dot_general(xb, wg_ref[0].astype(jnp.bfloat16), dn,
                               preferred_element_type=jnp.float32)
    up = jax.lax.dot_general(xb, wu_ref[0].astype(jnp.bfloat16), dn,
                             preferred_element_type=jnp.float32)
    h = (gate * jax.nn.sigmoid(gate)) * up

    e_prev = k // 2 - 1
    hprev = hb_s[1 - par]
    d = jax.lax.dot_general(hprev, wd_ref[0].astype(jnp.bfloat16), dn,
                            preferred_element_type=jnp.float32)
    wc = w_s[...]
    onehot = (jax.lax.broadcasted_iota(jnp.int32, wc.shape, 1) == e_prev)
    wcol = jnp.sum(wc * onehot.astype(jnp.float32), axis=1, keepdims=True)
    ocols = pl.ds(half * DS, DS)
    out_ref[:, ocols] = jnp.where(
        k <= 1, 0.0, out_ref[:, ocols] + d * wcol)

    hb_s[par, :, pl.ds(half * FS, FS)] = h.astype(jnp.bfloat16)


def kernel(x, Wr, Wg, Wu, Wd, temperature):
    B, T, D = x.shape
    E, F, _ = Wg.shape
    FS = F // 2
    DS = D // 2
    n_steps = 2 * E + 2
    x2 = x.reshape(B * T, D)

    def _gu_idx(k):
        kc = jnp.minimum(k, 2 * E - 1)
        return (kc // 2, kc % 2, 0)

    def _wd_idx(k):
        e_prev = jnp.clip(k // 2 - 1, 0, E - 1)
        return (e_prev, k % 2, 0)

    out = pl.pallas_call(
        _moe_body,
        grid=(n_steps,),
        in_specs=[
            pl.BlockSpec(memory_space=pltpu.SMEM),
            pl.BlockSpec((B * T, D), lambda k: (0, 0)),
            pl.BlockSpec((E, D), lambda k: (0, 0)),
            pl.BlockSpec((1, FS, D), _gu_idx),
            pl.BlockSpec((1, FS, D), _gu_idx),
            pl.BlockSpec((1, DS, F), _wd_idx),
        ],
        out_specs=pl.BlockSpec((B * T, D), lambda k: (0, 0)),
        out_shape=jax.ShapeDtypeStruct((B * T, D), jnp.float32),
        scratch_shapes=[
            pltpu.VMEM((B * T, D), jnp.bfloat16),
            pltpu.VMEM((2, B * T, F), jnp.bfloat16),
            pltpu.VMEM((B * T, E), jnp.float32),
        ],
    )(temperature, x2, Wr, Wg, Wu, Wd)
    return out.reshape(B, T, D).astype(x.dtype)


# speedup vs baseline: 1.1674x; 1.1674x over previous
import math

import jax
import jax.numpy as jnp
from jax.experimental import pallas as pl
from jax.experimental.pallas import tpu as pltpu

_GOLDEN_CENTER = 1 / math.e
_GOLDEN_LOWER = 0.5 - math.log(4 / 3)
_GOLDEN_UPPER = 0.5

_NF = 4


def _moe_body(temp_ref, x_ref, wr_ref, wg_ref, wu_ref, wd_ref, out_ref,
              xb_s, hb_s, wdb_s, w_s):
    e = pl.program_id(0)
    f = pl.program_id(1)
    E = wr_ref.shape[0]
    FS = wg_ref.shape[1]
    F = FS * pl.num_programs(1)
    fcols = pl.ds(f * FS, FS)
    dn = (((1,), (1,)), ((), ()))

    @pl.when(jnp.logical_and(e == 0, f == 0))
    def _router_and_xcast():
        xc = x_ref[...]
        xb_s[...] = xc.astype(jnp.bfloat16)
        logits = jax.lax.dot_general(
            xc, wr_ref[...], dn,
            preferred_element_type=jnp.float32)
        inhib = jax.nn.sigmoid(logits / temp_ref[0])
        in_zone = jnp.logical_and(inhib >= _GOLDEN_LOWER, inhib <= _GOLDEN_UPPER)
        dist = jnp.abs(inhib - _GOLDEN_CENTER)
        w = jnp.exp(-dist / 0.1) * in_zone.astype(jnp.float32)
        wsum = jnp.sum(w, axis=1, keepdims=True)
        no_expert = wsum < 1e-8
        fb = jnp.exp(-dist / 0.3)
        idx = jax.lax.broadcasted_iota(jnp.int32, fb.shape, 1)
        m1 = jnp.max(fb, axis=1, keepdims=True)
        i1 = jnp.min(jnp.where(fb == m1, idx, E), axis=1, keepdims=True)
        mask1 = idx == i1
        fb2 = jnp.where(mask1, -jnp.inf, fb)
        m2 = jnp.max(fb2, axis=1, keepdims=True)
        i2 = jnp.min(jnp.where(fb2 == m2, idx, E), axis=1, keepdims=True)
        fbm = jnp.logical_or(mask1, idx == i2).astype(jnp.float32)
        fb_w = fb * fbm
        fb_w = fb_w / jnp.maximum(jnp.sum(fb_w, axis=1, keepdims=True), 1e-8)
        w = jnp.where(no_expert, fb_w, w)
        w = w / jnp.maximum(jnp.sum(w, axis=1, keepdims=True), 1e-8)
        w_s[...] = w

    def _gate_up_stash():
        xb = xb_s[...]
        gate = jax.lax.dot_general(xb, wg_ref[0].astype(jnp.bfloat16), dn,
                                   preferred_element_type=jnp.float32)
        up = jax.lax.dot_general(xb, wu_ref[0].astype(jnp.bfloat16), dn,
                                 preferred_element_type=jnp.float32)
        h = (gate * jax.nn.sigmoid(gate)) * up
        hb_s[:, fcols] = h.astype(jnp.bfloat16)
        wdb_s[:, fcols] = wd_ref[0, :, fcols].astype(jnp.bfloat16)

    @pl.when(f < _NF - 1)
    def _produce():
        _gate_up_stash()

    @pl.when(f == _NF - 1)
    def _produce_down_accum():
        _gate_up_stash()
        half = F // 2
        d0 = jax.lax.dot_general(hb_s[:, pl.ds(0, half)],
                                 wdb_s[:, pl.ds(0, half)], dn,
                                 preferred_element_type=jnp.float32)
        d1 = jax.lax.dot_general(hb_s[:, pl.ds(half, half)],
                                 wdb_s[:, pl.ds(half, half)], dn,
                                 preferred_element_type=jnp.float32)
        wc = w_s[...]
        onehot = (jax.lax.broadcasted_iota(jnp.int32, wc.shape, 1) == e)
        wcol = jnp.sum(wc * onehot.astype(jnp.float32), axis=1, keepdims=True)
        contrib = (d0 + d1) * wcol
        out_ref[...] = jnp.where(e == 0, 0.0, out_ref[...]) + contrib


def kernel(x, Wr, Wg, Wu, Wd, temperature):
    B, T, D = x.shape
    E, F, _ = Wg.shape
    FS = F // _NF
    x2 = x.reshape(B * T, D)

    out = pl.pallas_call(
        _moe_body,
        grid=(E, _NF),
        in_specs=[
            pl.BlockSpec(memory_space=pltpu.SMEM),
            pl.BlockSpec((B * T, D), lambda e, f: (0, 0)),
            pl.BlockSpec((E, D), lambda e, f: (0, 0)),
            pl.BlockSpec((1, FS, D), lambda e, f: (e, f, 0)),
            pl.BlockSpec((1, FS, D), lambda e, f: (e, f, 0)),
            pl.BlockSpec((1, D, F), lambda e, f: (e, 0, 0)),
        ],
        out_specs=pl.BlockSpec((B * T, D), lambda e, f: (0, 0)),
        out_shape=jax.ShapeDtypeStruct((B * T, D), jnp.float32),
        scratch_shapes=[
            pltpu.VMEM((B * T, D), jnp.bfloat16),
            pltpu.VMEM((B * T, F), jnp.bfloat16),
            pltpu.VMEM((D, F), jnp.bfloat16),
            pltpu.VMEM((B * T, E), jnp.float32),
        ],
    )(temperature, x2, Wr, Wg, Wu, Wd)
    return out.reshape(B, T, D).astype(x.dtype)
